# Initial kernel scaffold; baseline (speedup 1.0000x reference)
import jax
import jax.numpy as jnp
from jax.experimental import pallas as pl


def kernel(points, features, lorentz_vectors, mask, stage1_w, stage2_Wa, stage2_ba, stage2_Wb, rr_W1, rr_b1, rr_W2, rr_b2):
    raise NotImplementedError("write your pallas kernel here")


# speedup vs baseline: 5.9109x; 5.9109x over previous
import numpy as np
import jax
import jax.numpy as jnp
from jax.experimental import pallas as pl

B, C, N = 128, 32, 1024
K1, K2 = 256, 50
H1, H2 = 128, 256
NP = 1225
NPP = 1280
M_TAU_C = 1.777
NEG = -1e9
MIN32 = -2147483648

LA = 8192
BB = 32
BBB = 4

_ii, _jj = np.triu_indices(K2, k=1)
_GIP = np.zeros((K2, NPP), np.float32)
_GJP = np.zeros((K2, NPP), np.float32)
_GIP[:, :NP] = np.arange(K2)[:, None] == _ii[None, :]
_GJP[:, :NP] = np.arange(K2)[:, None] == _jj[None, :]


def _scores_body(f_ref, w1_ref, waT_ref, ba_ref, wbT_ref, s1_ref, s2_ref):
    F = f_ref[...]
    w1t = w1_ref[...].reshape(1, C)
    s1 = jnp.dot(w1t.astype(jnp.bfloat16), F.astype(jnp.bfloat16),
                 preferred_element_type=jnp.float32)
    x = jnp.concatenate([F, s1], axis=0)
    h = jnp.dot(waT_ref[...].astype(jnp.bfloat16), x.astype(jnp.bfloat16),
                preferred_element_type=jnp.float32)
    h = jnp.maximum(h + ba_ref[...], 0.0)
    s2 = jnp.dot(wbT_ref[...].astype(jnp.bfloat16), h.astype(jnp.bfloat16),
                 preferred_element_type=jnp.float32)
    s1_ref[...] = s1
    s2_ref[...] = s2


def _select_body(s1_ref, s2_ref, m_ref, idx_ref, s2m_ref):
    s1 = s1_ref[...]
    s2 = s2_ref[...]
    msk = m_ref[...] > 0.5
    s1m = jnp.where(msk, s1, NEG)
    ib = jax.lax.bitcast_convert_type(s1m, jnp.int32)
    key = jnp.where(ib >= 0, ib,
                    jnp.bitwise_xor(jnp.bitwise_not(ib), MIN32))

    def radix_step(t, pu):
        bm = jnp.left_shift(jnp.int32(1), 31 - t)
        cand_u = jnp.bitwise_or(pu, bm)
        cand_s = jnp.bitwise_xor(cand_u, MIN32)
        cnt = jnp.sum((key >= cand_s).astype(jnp.int32), axis=1,
                      keepdims=True)
        return jnp.where(cnt >= K1, cand_u, pu)

    pu = jax.lax.fori_loop(0, 32, radix_step,
                           jnp.zeros((BB, 1), jnp.int32))
    thr = jnp.bitwise_xor(pu, MIN32)
    cond = jnp.logical_and(msk, key >= thr)
    s2m = jnp.where(cond, s2, NEG)
    s2m_ref[...] = s2m

    lanesN = jax.lax.broadcasted_iota(jnp.int32, (BB, N), 1)
    lane50 = jax.lax.broadcasted_iota(jnp.int32, (BB, K2), 1)

    def topk_step(k, carry):
        cur, acc = carry
        v = jnp.max(cur, axis=1, keepdims=True)
        isel = jnp.min(jnp.where(cur == v, lanesN, N), axis=1,
                       keepdims=True)
        acc = jnp.where(lane50 == k, isel, acc)
        cur = jnp.where(lanesN == isel, -3.0e38, cur)
        return cur, acc

    _, idx = jax.lax.fori_loop(
        0, K2, topk_step, (s2m, jnp.zeros((BB, K2), jnp.int32)))
    idx_ref[...] = idx


def _couple_body(f_ref, p_ref, l_ref, s1_ref, s2m_ref, idx_ref,
                 gi_ref, gj_ref, w1T_ref, b1_ref, w2T_ref, b2_ref,
                 out_ref, fm_ref):
    Gi = gi_ref[...]
    Gj = gj_ref[...]
    F20 = f_ref[...][0:20, :]
    P = p_ref[...]
    L = l_ref[...]
    s1f = s1_ref[...]
    s2f = s2m_ref[...]
    idx3 = idx_ref[...]
    iota_col = jax.lax.broadcasted_iota(jnp.int32, (N, K2), 0)
    cfs = []
    fms = []
    for e in range(BBB):
        sl = slice(e * N, (e + 1) * N)
        data = jnp.concatenate(
            [F20[:, sl], P[:, sl], L[:, sl], s1f[:, sl], s2f[:, sl]],
            axis=0)
        idx_e = jax.lax.slice(idx3, (0, e, 0), (1, e + 1, K2))
        idx_e = idx_e.reshape(1, K2)
        ohT = (iota_col == idx_e).astype(jnp.float32)
        g = jnp.dot(data, ohT, preferred_element_type=jnp.float32,
                    precision=jax.lax.Precision.HIGHEST)
        ci = jnp.dot(g, Gi, preferred_element_type=jnp.float32,
                     precision=jax.lax.Precision.HIGHEST)
        cj = jnp.dot(g, Gj, preferred_element_type=jnp.float32,
                     precision=jax.lax.Precision.HIGHEST)
        pi = ci[20:22, :]
        pj = cj[20:22, :]
        ps = ci[22:26, :] + cj[22:26, :]
        s2i = ci[27:28, :]
        s2j = cj[27:28, :]
        m2 = ps[3:4, :] ** 2 - (ps[0:1, :] ** 2 + ps[1:2, :] ** 2
                                + ps[2:3, :] ** 2)
        m = jnp.sqrt(jnp.maximum(m2, 0.0))
        pt = jnp.sqrt(ps[0:1, :] ** 2 + ps[1:2, :] ** 2)
        d = pi - pj
        dR = jnp.sqrt(d[0:1, :] ** 2 + d[1:2, :] ** 2)
        cf = jnp.concatenate(
            [ci[0:20, :], cj[0:20, :], pi, pj,
             ci[26:27, :], cj[26:27, :], s2i, s2j, m, pt, dR],
            axis=0)
        fm = jnp.logical_and(
            m <= M_TAU_C,
            jnp.logical_and(s2i > -1e8, s2j > -1e8)).astype(jnp.float32)
        cfs.append(cf)
        fms.append(fm)
    cf_cat = jnp.concatenate(cfs, axis=1)
    hc = jnp.dot(w1T_ref[...].astype(jnp.bfloat16),
                 cf_cat.astype(jnp.bfloat16),
                 preferred_element_type=jnp.float32)
    hc = jnp.maximum(hc + b1_ref[...], 0.0)
    sc = jnp.dot(w2T_ref[...].astype(jnp.bfloat16),
                 hc.astype(jnp.bfloat16),
                 preferred_element_type=jnp.float32)
    sc = sc + b2_ref[...]
    for e in range(BBB):
        out_ref[e, :, :] = sc[:, e * NPP:e * NPP + NP]
        fm_ref[e, :, :] = fms[e][:, :NP]


def kernel(points, features, lorentz_vectors, mask, stage1_w, stage2_Wa,
           stage2_ba, stage2_Wb, rr_W1, rr_b1, rr_W2, rr_b2):
    F2D = features.transpose(1, 0, 2).reshape(C, B * N)
    P2D = points.transpose(1, 0, 2).reshape(2, B * N)
    L2D = lorentz_vectors.transpose(1, 0, 2).reshape(4, B * N)
    maskBN = mask.reshape(B, N)
    w1c = stage1_w.reshape(C, 1)
    WaT = stage2_Wa.T
    bac = stage2_ba.reshape(H1, 1)
    WbT = stage2_Wb.T
    W1T = rr_W1.T
    b1c = rr_b1.reshape(H2, 1)
    W2T = rr_W2.T
    b2c = rr_b2.reshape(1, 1)
    Gi = jnp.asarray(_GIP)
    Gj = jnp.asarray(_GJP)

    full = lambda shape: pl.BlockSpec(shape, lambda i: tuple(0 for _ in shape))

    s1f, s2f = pl.pallas_call(
        _scores_body,
        grid=(B * N // LA,),
        in_specs=[
            pl.BlockSpec((C, LA), lambda i: (0, i)),
            full((C, 1)), full((H1, C + 1)), full((H1, 1)), full((1, H1)),
        ],
        out_specs=[
            pl.BlockSpec((1, LA), lambda i: (0, i)),
            pl.BlockSpec((1, LA), lambda i: (0, i)),
        ],
        out_shape=[
            jax.ShapeDtypeStruct((1, B * N), jnp.float32),
            jax.ShapeDtypeStruct((1, B * N), jnp.float32),
        ],
    )(F2D, w1c, WaT, bac, WbT)

    idx2, s2m = pl.pallas_call(
        _select_body,
        grid=(B // BB,),
        in_specs=[
            pl.BlockSpec((BB, N), lambda i: (i, 0)),
            pl.BlockSpec((BB, N), lambda i: (i, 0)),
            pl.BlockSpec((BB, N), lambda i: (i, 0)),
        ],
        out_specs=[
            pl.BlockSpec((BB, K2), lambda i: (i, 0)),
            pl.BlockSpec((BB, N), lambda i: (i, 0)),
        ],
        out_shape=[
            jax.ShapeDtypeStruct((B, K2), jnp.int32),
            jax.ShapeDtypeStruct((B, N), jnp.float32),
        ],
    )(s1f.reshape(B, N), s2f.reshape(B, N), maskBN)

    scores3, fm3 = pl.pallas_call(
        _couple_body,
        grid=(B // BBB,),
        in_specs=[
            pl.BlockSpec((C, BBB * N), lambda i: (0, i)),
            pl.BlockSpec((2, BBB * N), lambda i: (0, i)),
            pl.BlockSpec((4, BBB * N), lambda i: (0, i)),
            pl.BlockSpec((1, BBB * N), lambda i: (0, i)),
            pl.BlockSpec((1, BBB * N), lambda i: (0, i)),
            pl.BlockSpec((1, BBB, K2), lambda i: (i, 0, 0)),
            full((K2, NPP)), full((K2, NPP)),
            full((H2, 51)), full((H2, 1)), full((1, H2)), full((1, 1)),
        ],
        out_specs=[
            pl.BlockSpec((BBB, 1, NP), lambda i: (i, 0, 0)),
            pl.BlockSpec((BBB, 1, NP), lambda i: (i, 0, 0)),
        ],
        out_shape=[
            jax.ShapeDtypeStruct((B, 1, NP), jnp.float32),
            jax.ShapeDtypeStruct((B, 1, NP), jnp.float32),
        ],
    )(F2D, P2D, L2D, s1f, s2m.reshape(1, B * N),
      idx2.reshape(B // BBB, BBB, K2), Gi, Gj, W1T, b1c, W2T, b2c)

    return scores3.reshape(B, NP), fm3.reshape(B, NP) != 0.0


# speedup vs baseline: 6.6416x; 1.1236x over previous
import numpy as np
import jax
import jax.numpy as jnp
from jax.experimental import pallas as pl

B, C, N = 128, 32, 1024
K1, K2 = 256, 50
H1, H2 = 128, 256
NP = 1225
NPP = 1280
M_TAU_C = 1.777
NEG = -1e9
MIN32 = -2147483648

BA = 8
BB = 32
BBB = 8

_ii, _jj = np.triu_indices(K2, k=1)
_GIP = np.zeros((K2, NPP), np.float32)
_GJP = np.zeros((K2, NPP), np.float32)
_GIP[:, :NP] = np.arange(K2)[:, None] == _ii[None, :]
_GJP[:, :NP] = np.arange(K2)[:, None] == _jj[None, :]


def _scores_body(f_ref, w1_ref, waT_ref, ba_ref, wbT_ref, s1_ref, s2_ref):
    w1t = w1_ref[...].reshape(1, C)
    for e in range(BA):
        F = f_ref[e]
        s1 = jnp.dot(w1t.astype(jnp.bfloat16), F.astype(jnp.bfloat16),
                     preferred_element_type=jnp.float32)
        x = jnp.concatenate([F, s1], axis=0)
        h = jnp.dot(waT_ref[...].astype(jnp.bfloat16),
                    x.astype(jnp.bfloat16),
                    preferred_element_type=jnp.float32)
        h = jnp.maximum(h + ba_ref[...], 0.0)
        s2 = jnp.dot(wbT_ref[...].astype(jnp.bfloat16),
                     h.astype(jnp.bfloat16),
                     preferred_element_type=jnp.float32)
        s1_ref[pl.ds(e, 1), :] = s1
        s2_ref[pl.ds(e, 1), :] = s2


def _sortable(x):
    ib = jax.lax.bitcast_convert_type(x, jnp.int32)
    return jnp.where(ib >= 0, ib,
                     jnp.bitwise_xor(jnp.bitwise_not(ib), MIN32))


def _radix_thr(key, k_want):
    def radix_step(t, pu):
        bm = jnp.left_shift(jnp.int32(1), 31 - t)
        cand_u = jnp.bitwise_or(pu, bm)
        cand_s = jnp.bitwise_xor(cand_u, MIN32)
        cnt = jnp.sum((key >= cand_s).astype(jnp.int32), axis=1,
                      keepdims=True)
        return jnp.where(cnt >= k_want, cand_u, pu)

    pu = jax.lax.fori_loop(0, 32, radix_step,
                           jnp.zeros((key.shape[0], 1), jnp.int32))
    return jnp.bitwise_xor(pu, MIN32)


def _select_body(s1_ref, s2_ref, m_ref, idx_ref, s2m_ref):
    s1 = s1_ref[...]
    s2 = s2_ref[...]
    msk = m_ref[...] > 0.5
    s1m = jnp.where(msk, s1, NEG)
    key1 = _sortable(s1m)
    thr1 = _radix_thr(key1, K1)
    cond1 = jnp.logical_and(msk, key1 >= thr1)
    s2m = jnp.where(cond1, s2, NEG)
    s2m_ref[...] = s2m

    lanesN = jax.lax.broadcasted_iota(jnp.int32, (BB, N), 1)
    lane50 = jax.lax.broadcasted_iota(jnp.int32, (BB, K2), 1)

    def topk_step(k, carry):
        cur, acc = carry
        isel = jnp.argmax(cur, axis=1).astype(jnp.int32)[:, None]
        acc = jnp.where(lane50 == k, isel, acc)
        cur = jnp.where(lanesN == isel, -3.0e38, cur)
        return cur, acc

    _, idx = jax.lax.fori_loop(
        0, K2, topk_step, (s2m, jnp.zeros((BB, K2), jnp.int32)))
    idx_ref[...] = idx


def _couple_body(f_ref, p_ref, l_ref, s1_ref, s2m_ref, idx_ref,
                 gi_ref, gj_ref, w1T_ref, b1_ref, w2T_ref, b2_ref,
                 out_ref, fm_ref):
    Gi = gi_ref[...]
    Gj = gj_ref[...]
    s1b = s1_ref[...]
    s2b = s2m_ref[...]
    idxb = idx_ref[...]
    iota_col = jax.lax.broadcasted_iota(jnp.int32, (N, K2), 0)
    cfs = []
    fms = []
    for e in range(BBB):
        data = jnp.concatenate(
            [f_ref[e][0:20, :], p_ref[e], l_ref[e],
             s1b[e:e + 1, :], s2b[e:e + 1, :]], axis=0)
        idx_e = idxb[e:e + 1, :]
        ohT = (iota_col == idx_e).astype(jnp.float32)
        g = jnp.dot(data, ohT, preferred_element_type=jnp.float32,
                    precision=jax.lax.Precision.HIGHEST)
        ci = jnp.dot(g, Gi, preferred_element_type=jnp.float32,
                     precision=jax.lax.Precision.HIGHEST)
        cj = jnp.dot(g, Gj, preferred_element_type=jnp.float32,
                     precision=jax.lax.Precision.HIGHEST)
        pi = ci[20:22, :]
        pj = cj[20:22, :]
        ps = ci[22:26, :] + cj[22:26, :]
        s2i = ci[27:28, :]
        s2j = cj[27:28, :]
        m2 = ps[3:4, :] ** 2 - (ps[0:1, :] ** 2 + ps[1:2, :] ** 2
                                + ps[2:3, :] ** 2)
        m = jnp.sqrt(jnp.maximum(m2, 0.0))
        pt = jnp.sqrt(ps[0:1, :] ** 2 + ps[1:2, :] ** 2)
        d = pi - pj
        dR = jnp.sqrt(d[0:1, :] ** 2 + d[1:2, :] ** 2)
        cf = jnp.concatenate(
            [ci[0:20, :], cj[0:20, :], pi, pj,
             ci[26:27, :], cj[26:27, :], s2i, s2j, m, pt, dR],
            axis=0)
        fm = jnp.logical_and(
            m <= M_TAU_C,
            jnp.logical_and(s2i > -1e8, s2j > -1e8)).astype(jnp.float32)
        cfs.append(cf)
        fms.append(fm)
    cf_cat = jnp.concatenate(cfs, axis=1)
    hc = jnp.dot(w1T_ref[...].astype(jnp.bfloat16),
                 cf_cat.astype(jnp.bfloat16),
                 preferred_element_type=jnp.float32)
    hc = jnp.maximum(hc + b1_ref[...], 0.0)
    sc = jnp.dot(w2T_ref[...].astype(jnp.bfloat16),
                 hc.astype(jnp.bfloat16),
                 preferred_element_type=jnp.float32)
    sc = sc + b2_ref[...]
    for e in range(BBB):
        out_ref[pl.ds(e, 1), :] = sc[:, e * NPP:e * NPP + NP]
        fm_ref[pl.ds(e, 1), :] = fms[e][:, :NP]


def kernel(points, features, lorentz_vectors, mask, stage1_w, stage2_Wa,
           stage2_ba, stage2_Wb, rr_W1, rr_b1, rr_W2, rr_b2):
    maskBN = mask.reshape(B, N)
    w1c = stage1_w.reshape(C, 1)
    WaT = stage2_Wa.T
    bac = stage2_ba.reshape(H1, 1)
    WbT = stage2_Wb.T
    W1T = rr_W1.T
    b1c = rr_b1.reshape(H2, 1)
    W2T = rr_W2.T
    b2c = rr_b2.reshape(1, 1)
    Gi = jnp.asarray(_GIP)
    Gj = jnp.asarray(_GJP)

    full = lambda shape: pl.BlockSpec(shape, lambda i: tuple(0 for _ in shape))

    s1f, s2f = pl.pallas_call(
        _scores_body,
        grid=(B // BA,),
        in_specs=[
            pl.BlockSpec((BA, C, N), lambda i: (i, 0, 0)),
            full((C, 1)), full((H1, C + 1)), full((H1, 1)), full((1, H1)),
        ],
        out_specs=[
            pl.BlockSpec((BA, N), lambda i: (i, 0)),
            pl.BlockSpec((BA, N), lambda i: (i, 0)),
        ],
        out_shape=[
            jax.ShapeDtypeStruct((B, N), jnp.float32),
            jax.ShapeDtypeStruct((B, N), jnp.float32),
        ],
    )(features, w1c, WaT, bac, WbT)

    idx2, s2m = pl.pallas_call(
        _select_body,
        grid=(B // BB,),
        in_specs=[
            pl.BlockSpec((BB, N), lambda i: (i, 0)),
            pl.BlockSpec((BB, N), lambda i: (i, 0)),
            pl.BlockSpec((BB, N), lambda i: (i, 0)),
        ],
        out_specs=[
            pl.BlockSpec((BB, K2), lambda i: (i, 0)),
            pl.BlockSpec((BB, N), lambda i: (i, 0)),
        ],
        out_shape=[
            jax.ShapeDtypeStruct((B, K2), jnp.int32),
            jax.ShapeDtypeStruct((B, N), jnp.float32),
        ],
    )(s1f, s2f, maskBN)

    scores, fmf = pl.pallas_call(
        _couple_body,
        grid=(B // BBB,),
        in_specs=[
            pl.BlockSpec((BBB, C, N), lambda i: (i, 0, 0)),
            pl.BlockSpec((BBB, 2, N), lambda i: (i, 0, 0)),
            pl.BlockSpec((BBB, 4, N), lambda i: (i, 0, 0)),
            pl.BlockSpec((BBB, N), lambda i: (i, 0)),
            pl.BlockSpec((BBB, N), lambda i: (i, 0)),
            pl.BlockSpec((BBB, K2), lambda i: (i, 0)),
            full((K2, NPP)), full((K2, NPP)),
            full((H2, 51)), full((H2, 1)), full((1, H2)), full((1, 1)),
        ],
        out_specs=[
            pl.BlockSpec((BBB, NP), lambda i: (i, 0)),
            pl.BlockSpec((BBB, NP), lambda i: (i, 0)),
        ],
        out_shape=[
            jax.ShapeDtypeStruct((B, NP), jnp.float32),
            jax.ShapeDtypeStruct((B, NP), jnp.float32),
        ],
    )(features, points, lorentz_vectors, s1f, s2m, idx2,
      Gi, Gj, W1T, b1c, W2T, b2c)

    return scores, fmf != 0.0


# speedup vs baseline: 8.5722x; 1.2907x over previous
import numpy as np
import jax
import jax.numpy as jnp
from jax.experimental import pallas as pl

B, C, N = 128, 32, 1024
K1, K2 = 256, 50
H1, H2 = 128, 256
NP = 1225
NPP = 1280
M_TAU_C = 1.777
NEG = -1e9
MIN32 = -2147483648

BA = 8
BB = 64
BBB = 8


def _split3(a):
    hi = a.astype(jnp.bfloat16)
    r1 = a - hi.astype(jnp.float32)
    mid = r1.astype(jnp.bfloat16)
    lo = (r1 - mid.astype(jnp.float32)).astype(jnp.bfloat16)
    return hi, mid, lo


def _gather_dot(a, ohT_bf16):
    hi, mid, lo = _split3(a)
    acc = jnp.dot(hi, ohT_bf16, preferred_element_type=jnp.float32)
    acc = acc + jnp.dot(mid, ohT_bf16, preferred_element_type=jnp.float32)
    return acc + jnp.dot(lo, ohT_bf16, preferred_element_type=jnp.float32)

_ii, _jj = np.triu_indices(K2, k=1)
_GIP = np.zeros((K2, NPP), np.float32)
_GJP = np.zeros((K2, NPP), np.float32)
_GIP[:, :NP] = np.arange(K2)[:, None] == _ii[None, :]
_GJP[:, :NP] = np.arange(K2)[:, None] == _jj[None, :]


def _scores_body(f_ref, w1_ref, waT_ref, ba_ref, wbT_ref, s1_ref, s2_ref):
    w1t = w1_ref[...].reshape(1, C)
    for e in range(BA):
        F = f_ref[e]
        s1 = jnp.dot(w1t.astype(jnp.bfloat16), F.astype(jnp.bfloat16),
                     preferred_element_type=jnp.float32)
        x = jnp.concatenate([F, s1], axis=0)
        h = jnp.dot(waT_ref[...].astype(jnp.bfloat16),
                    x.astype(jnp.bfloat16),
                    preferred_element_type=jnp.float32)
        h = jnp.maximum(h + ba_ref[...], 0.0)
        s2 = jnp.dot(wbT_ref[...].astype(jnp.bfloat16),
                     h.astype(jnp.bfloat16),
                     preferred_element_type=jnp.float32)
        s1_ref[pl.ds(e, 1), :] = s1
        s2_ref[pl.ds(e, 1), :] = s2


def _sortable(x):
    ib = jax.lax.bitcast_convert_type(x, jnp.int32)
    return jnp.where(ib >= 0, ib,
                     jnp.bitwise_xor(jnp.bitwise_not(ib), MIN32))


def _radix_thr(key, k_want):
    def radix_step(t, pu):
        bm = jnp.left_shift(jnp.int32(1), 31 - t)
        cand_u = jnp.bitwise_or(pu, bm)
        cand_s = jnp.bitwise_xor(cand_u, MIN32)
        cnt = jnp.sum((key >= cand_s).astype(jnp.int32), axis=1,
                      keepdims=True)
        return jnp.where(cnt >= k_want, cand_u, pu)

    pu = jax.lax.fori_loop(0, 32, radix_step,
                           jnp.zeros((key.shape[0], 1), jnp.int32))
    return jnp.bitwise_xor(pu, MIN32)


def _select_body(s1_ref, s2_ref, m_ref, idx_ref, s2m_ref):
    s1 = s1_ref[...]
    s2 = s2_ref[...]
    msk = m_ref[...] > 0.5
    s1m = jnp.where(msk, s1, NEG)
    key1 = _sortable(s1m)
    thr1 = _radix_thr(key1, K1)
    cond1 = jnp.logical_and(msk, key1 >= thr1)
    s2m = jnp.where(cond1, s2, NEG)
    s2m_ref[...] = s2m

    lanesN = jax.lax.broadcasted_iota(jnp.int32, (BB, N), 1)
    lane50 = jax.lax.broadcasted_iota(jnp.int32, (BB, K2), 1)

    def topk_step(k, carry):
        cur, acc = carry
        isel = jnp.argmax(cur, axis=1).astype(jnp.int32)[:, None]
        acc = jnp.where(lane50 == k, isel, acc)
        cur = jnp.where(lanesN == isel, -3.0e38, cur)
        return cur, acc

    _, idx = jax.lax.fori_loop(
        0, K2, topk_step, (s2m, jnp.zeros((BB, K2), jnp.int32)))
    idx_ref[...] = idx


def _couple_body(f_ref, p_ref, l_ref, s1_ref, s2m_ref, idx_ref,
                 gi_ref, gj_ref, w1T_ref, b1_ref, w2T_ref, b2_ref,
                 out_ref, fm_ref):
    Gi = gi_ref[...]
    Gj = gj_ref[...]
    s1b = s1_ref[...]
    s2b = s2m_ref[...]
    idxb = idx_ref[...]
    iota_col = jax.lax.broadcasted_iota(jnp.int32, (N, K2), 0)
    cfs = []
    fms = []
    for e in range(BBB):
        data = jnp.concatenate(
            [f_ref[e][0:20, :], p_ref[e], l_ref[e],
             s1b[e:e + 1, :], s2b[e:e + 1, :]], axis=0)
        idx_e = idxb[e:e + 1, :]
        ohT = (iota_col == idx_e).astype(jnp.bfloat16)
        g = _gather_dot(data, ohT)
        ci = _gather_dot(g, Gi)
        cj = _gather_dot(g, Gj)
        pi = ci[20:22, :]
        pj = cj[20:22, :]
        ps = ci[22:26, :] + cj[22:26, :]
        s2i = ci[27:28, :]
        s2j = cj[27:28, :]
        m2 = ps[3:4, :] ** 2 - (ps[0:1, :] ** 2 + ps[1:2, :] ** 2
                                + ps[2:3, :] ** 2)
        m = jnp.sqrt(jnp.maximum(m2, 0.0))
        pt = jnp.sqrt(ps[0:1, :] ** 2 + ps[1:2, :] ** 2)
        d = pi - pj
        dR = jnp.sqrt(d[0:1, :] ** 2 + d[1:2, :] ** 2)
        cf = jnp.concatenate(
            [ci[0:20, :], cj[0:20, :], pi, pj,
             ci[26:27, :], cj[26:27, :], s2i, s2j, m, pt, dR],
            axis=0)
        fm = jnp.logical_and(
            m <= M_TAU_C,
            jnp.logical_and(s2i > -1e8, s2j > -1e8)).astype(jnp.float32)
        cfs.append(cf)
        fms.append(fm)
    cf_cat = jnp.concatenate(cfs, axis=1)
    hc = jnp.dot(w1T_ref[...].astype(jnp.bfloat16),
                 cf_cat.astype(jnp.bfloat16),
                 preferred_element_type=jnp.float32)
    hc = jnp.maximum(hc + b1_ref[...], 0.0)
    sc = jnp.dot(w2T_ref[...].astype(jnp.bfloat16),
                 hc.astype(jnp.bfloat16),
                 preferred_element_type=jnp.float32)
    sc = sc + b2_ref[...]
    for e in range(BBB):
        out_ref[pl.ds(e, 1), :] = sc[:, e * NPP:e * NPP + NP]
        fm_ref[pl.ds(e, 1), :] = fms[e][:, :NP]


def kernel(points, features, lorentz_vectors, mask, stage1_w, stage2_Wa,
           stage2_ba, stage2_Wb, rr_W1, rr_b1, rr_W2, rr_b2):
    maskBN = mask.reshape(B, N)
    w1c = stage1_w.reshape(C, 1)
    WaT = stage2_Wa.T
    bac = stage2_ba.reshape(H1, 1)
    WbT = stage2_Wb.T
    W1T = rr_W1.T
    b1c = rr_b1.reshape(H2, 1)
    W2T = rr_W2.T
    b2c = rr_b2.reshape(1, 1)
    Gi = jnp.asarray(_GIP).astype(jnp.bfloat16)
    Gj = jnp.asarray(_GJP).astype(jnp.bfloat16)

    full = lambda shape: pl.BlockSpec(shape, lambda i: tuple(0 for _ in shape))

    s1f, s2f = pl.pallas_call(
        _scores_body,
        grid=(B // BA,),
        in_specs=[
            pl.BlockSpec((BA, C, N), lambda i: (i, 0, 0)),
            full((C, 1)), full((H1, C + 1)), full((H1, 1)), full((1, H1)),
        ],
        out_specs=[
            pl.BlockSpec((BA, N), lambda i: (i, 0)),
            pl.BlockSpec((BA, N), lambda i: (i, 0)),
        ],
        out_shape=[
            jax.ShapeDtypeStruct((B, N), jnp.float32),
            jax.ShapeDtypeStruct((B, N), jnp.float32),
        ],
    )(features, w1c, WaT, bac, WbT)

    idx2, s2m = pl.pallas_call(
        _select_body,
        grid=(B // BB,),
        in_specs=[
            pl.BlockSpec((BB, N), lambda i: (i, 0)),
            pl.BlockSpec((BB, N), lambda i: (i, 0)),
            pl.BlockSpec((BB, N), lambda i: (i, 0)),
        ],
        out_specs=[
            pl.BlockSpec((BB, K2), lambda i: (i, 0)),
            pl.BlockSpec((BB, N), lambda i: (i, 0)),
        ],
        out_shape=[
            jax.ShapeDtypeStruct((B, K2), jnp.int32),
            jax.ShapeDtypeStruct((B, N), jnp.float32),
        ],
    )(s1f, s2f, maskBN)

    scores, fmf = pl.pallas_call(
        _couple_body,
        grid=(B // BBB,),
        in_specs=[
            pl.BlockSpec((BBB, C, N), lambda i: (i, 0, 0)),
            pl.BlockSpec((BBB, 2, N), lambda i: (i, 0, 0)),
            pl.BlockSpec((BBB, 4, N), lambda i: (i, 0, 0)),
            pl.BlockSpec((BBB, N), lambda i: (i, 0)),
            pl.BlockSpec((BBB, N), lambda i: (i, 0)),
            pl.BlockSpec((BBB, K2), lambda i: (i, 0)),
            full((K2, NPP)), full((K2, NPP)),
            full((H2, 51)), full((H2, 1)), full((1, H2)), full((1, 1)),
        ],
        out_specs=[
            pl.BlockSpec((BBB, NP), lambda i: (i, 0)),
            pl.BlockSpec((BBB, NP), lambda i: (i, 0)),
        ],
        out_shape=[
            jax.ShapeDtypeStruct((B, NP), jnp.float32),
            jax.ShapeDtypeStruct((B, NP), jnp.float32),
        ],
    )(features, points, lorentz_vectors, s1f, s2m, idx2,
      Gi, Gj, W1T, b1c, W2T, b2c)

    return scores, fmf != 0.0


# speedup vs baseline: 8.6254x; 1.0062x over previous
import numpy as np
import jax
import jax.numpy as jnp
from jax.experimental import pallas as pl

B, C, N = 128, 32, 1024
K1, K2 = 256, 50
H1, H2 = 128, 256
NP = 1225
NPP = 1280
M_TAU_C = 1.777
NEG = -1e9
MIN32 = -2147483648

BA = 8
BB = 64
BBB = 16


def _split3(a):
    hi = a.astype(jnp.bfloat16)
    r1 = a - hi.astype(jnp.float32)
    mid = r1.astype(jnp.bfloat16)
    lo = (r1 - mid.astype(jnp.float32)).astype(jnp.bfloat16)
    return hi, mid, lo


def _gather_dot(a, ohT_bf16):
    hi, mid, lo = _split3(a)
    acc = jnp.dot(hi, ohT_bf16, preferred_element_type=jnp.float32)
    acc = acc + jnp.dot(mid, ohT_bf16, preferred_element_type=jnp.float32)
    return acc + jnp.dot(lo, ohT_bf16, preferred_element_type=jnp.float32)

_ii, _jj = np.triu_indices(K2, k=1)
_GIP = np.zeros((K2, NPP), np.float32)
_GJP = np.zeros((K2, NPP), np.float32)
_GIP[:, :NP] = np.arange(K2)[:, None] == _ii[None, :]
_GJP[:, :NP] = np.arange(K2)[:, None] == _jj[None, :]


def _scores_body(f_ref, w1_ref, waT_ref, ba_ref, wbT_ref, s1_ref, s2_ref):
    w1t = w1_ref[...].reshape(1, C)
    for e in range(BA):
        F = f_ref[e]
        s1 = jnp.dot(w1t.astype(jnp.bfloat16), F.astype(jnp.bfloat16),
                     preferred_element_type=jnp.float32)
        x = jnp.concatenate([F, s1], axis=0)
        h = jnp.dot(waT_ref[...].astype(jnp.bfloat16),
                    x.astype(jnp.bfloat16),
                    preferred_element_type=jnp.float32)
        h = jnp.maximum(h + ba_ref[...], 0.0)
        s2 = jnp.dot(wbT_ref[...].astype(jnp.bfloat16),
                     h.astype(jnp.bfloat16),
                     preferred_element_type=jnp.float32)
        s1_ref[pl.ds(e, 1), :] = s1
        s2_ref[pl.ds(e, 1), :] = s2


def _sortable(x):
    ib = jax.lax.bitcast_convert_type(x, jnp.int32)
    return jnp.where(ib >= 0, ib,
                     jnp.bitwise_xor(jnp.bitwise_not(ib), MIN32))


def _radix_thr(key, k_want):
    def radix_step(t, pu):
        bm = jnp.left_shift(jnp.int32(1), 31 - t)
        cand_u = jnp.bitwise_or(pu, bm)
        cand_s = jnp.bitwise_xor(cand_u, MIN32)
        cnt = jnp.sum((key >= cand_s).astype(jnp.int32), axis=1,
                      keepdims=True)
        return jnp.where(cnt >= k_want, cand_u, pu)

    pu = jax.lax.fori_loop(0, 32, radix_step,
                           jnp.zeros((key.shape[0], 1), jnp.int32))
    return jnp.bitwise_xor(pu, MIN32)


def _select_body(s1_ref, s2_ref, m_ref, idx_ref, s2m_ref):
    s1 = s1_ref[...]
    s2 = s2_ref[...]
    msk = m_ref[...] > 0.5
    s1m = jnp.where(msk, s1, NEG)
    key1 = _sortable(s1m)
    thr1 = _radix_thr(key1, K1)
    cond1 = jnp.logical_and(msk, key1 >= thr1)
    s2m = jnp.where(cond1, s2, NEG)
    s2m_ref[...] = s2m

    lanesN = jax.lax.broadcasted_iota(jnp.int32, (BB, N), 1)
    lane50 = jax.lax.broadcasted_iota(jnp.int32, (BB, K2), 1)

    def topk_step(k, carry):
        cur, acc = carry
        isel = jnp.argmax(cur, axis=1).astype(jnp.int32)[:, None]
        acc = jnp.where(lane50 == k, isel, acc)
        cur = jnp.where(lanesN == isel, -3.0e38, cur)
        return cur, acc

    _, idx = jax.lax.fori_loop(
        0, K2, topk_step, (s2m, jnp.zeros((BB, K2), jnp.int32)))
    idx_ref[...] = idx


def _couple_body(f_ref, p_ref, l_ref, s1_ref, s2m_ref, idx_ref,
                 gi_ref, gj_ref, w1T_ref, b1_ref, w2T_ref, b2_ref,
                 out_ref, fm_ref):
    Gi = gi_ref[...]
    Gj = gj_ref[...]
    s1b = s1_ref[...]
    s2b = s2m_ref[...]
    idxb = idx_ref[...]
    iota_col = jax.lax.broadcasted_iota(jnp.int32, (N, K2), 0)
    cfs = []
    fms = []
    for e in range(BBB):
        data = jnp.concatenate(
            [f_ref[e][0:20, :], p_ref[e], l_ref[e],
             s1b[e:e + 1, :], s2b[e:e + 1, :]], axis=0)
        idx_e = idxb[e:e + 1, :]
        ohT = (iota_col == idx_e).astype(jnp.bfloat16)
        g = _gather_dot(data, ohT)
        ci = _gather_dot(g, Gi)
        cj = _gather_dot(g, Gj)
        pi = ci[20:22, :]
        pj = cj[20:22, :]
        ps = ci[22:26, :] + cj[22:26, :]
        s2i = ci[27:28, :]
        s2j = cj[27:28, :]
        m2 = ps[3:4, :] ** 2 - (ps[0:1, :] ** 2 + ps[1:2, :] ** 2
                                + ps[2:3, :] ** 2)
        m = jnp.sqrt(jnp.maximum(m2, 0.0))
        pt = jnp.sqrt(ps[0:1, :] ** 2 + ps[1:2, :] ** 2)
        d = pi - pj
        dR = jnp.sqrt(d[0:1, :] ** 2 + d[1:2, :] ** 2)
        cf = jnp.concatenate(
            [ci[0:20, :], cj[0:20, :], pi, pj,
             ci[26:27, :], cj[26:27, :], s2i, s2j, m, pt, dR],
            axis=0)
        fm = jnp.logical_and(
            m <= M_TAU_C,
            jnp.logical_and(s2i > -1e8, s2j > -1e8)).astype(jnp.float32)
        cfs.append(cf)
        fms.append(fm)
    cf_cat = jnp.concatenate(cfs, axis=1)
    hc = jnp.dot(w1T_ref[...].astype(jnp.bfloat16),
                 cf_cat.astype(jnp.bfloat16),
                 preferred_element_type=jnp.float32)
    hc = jnp.maximum(hc + b1_ref[...], 0.0)
    sc = jnp.dot(w2T_ref[...].astype(jnp.bfloat16),
                 hc.astype(jnp.bfloat16),
                 preferred_element_type=jnp.float32)
    sc = sc + b2_ref[...]
    for e in range(BBB):
        out_ref[pl.ds(e, 1), :] = sc[:, e * NPP:e * NPP + NP]
        fm_ref[pl.ds(e, 1), :] = fms[e][:, :NP]


def kernel(points, features, lorentz_vectors, mask, stage1_w, stage2_Wa,
           stage2_ba, stage2_Wb, rr_W1, rr_b1, rr_W2, rr_b2):
    maskBN = mask.reshape(B, N)
    w1c = stage1_w.reshape(C, 1)
    WaT = stage2_Wa.T
    bac = stage2_ba.reshape(H1, 1)
    WbT = stage2_Wb.T
    W1T = rr_W1.T
    b1c = rr_b1.reshape(H2, 1)
    W2T = rr_W2.T
    b2c = rr_b2.reshape(1, 1)
    Gi = jnp.asarray(_GIP).astype(jnp.bfloat16)
    Gj = jnp.asarray(_GJP).astype(jnp.bfloat16)

    full = lambda shape: pl.BlockSpec(shape, lambda i: tuple(0 for _ in shape))

    s1f, s2f = pl.pallas_call(
        _scores_body,
        grid=(B // BA,),
        in_specs=[
            pl.BlockSpec((BA, C, N), lambda i: (i, 0, 0)),
            full((C, 1)), full((H1, C + 1)), full((H1, 1)), full((1, H1)),
        ],
        out_specs=[
            pl.BlockSpec((BA, N), lambda i: (i, 0)),
            pl.BlockSpec((BA, N), lambda i: (i, 0)),
        ],
        out_shape=[
            jax.ShapeDtypeStruct((B, N), jnp.float32),
            jax.ShapeDtypeStruct((B, N), jnp.float32),
        ],
    )(features, w1c, WaT, bac, WbT)

    idx2, s2m = pl.pallas_call(
        _select_body,
        grid=(B // BB,),
        in_specs=[
            pl.BlockSpec((BB, N), lambda i: (i, 0)),
            pl.BlockSpec((BB, N), lambda i: (i, 0)),
            pl.BlockSpec((BB, N), lambda i: (i, 0)),
        ],
        out_specs=[
            pl.BlockSpec((BB, K2), lambda i: (i, 0)),
            pl.BlockSpec((BB, N), lambda i: (i, 0)),
        ],
        out_shape=[
            jax.ShapeDtypeStruct((B, K2), jnp.int32),
            jax.ShapeDtypeStruct((B, N), jnp.float32),
        ],
    )(s1f, s2f, maskBN)

    scores, fmf = pl.pallas_call(
        _couple_body,
        grid=(B // BBB,),
        in_specs=[
            pl.BlockSpec((BBB, C, N), lambda i: (i, 0, 0)),
            pl.BlockSpec((BBB, 2, N), lambda i: (i, 0, 0)),
            pl.BlockSpec((BBB, 4, N), lambda i: (i, 0, 0)),
            pl.BlockSpec((BBB, N), lambda i: (i, 0)),
            pl.BlockSpec((BBB, N), lambda i: (i, 0)),
            pl.BlockSpec((BBB, K2), lambda i: (i, 0)),
            full((K2, NPP)), full((K2, NPP)),
            full((H2, 51)), full((H2, 1)), full((1, H2)), full((1, 1)),
        ],
        out_specs=[
            pl.BlockSpec((BBB, NP), lambda i: (i, 0)),
            pl.BlockSpec((BBB, NP), lambda i: (i, 0)),
        ],
        out_shape=[
            jax.ShapeDtypeStruct((B, NP), jnp.float32),
            jax.ShapeDtypeStruct((B, NP), jnp.float32),
        ],
    )(features, points, lorentz_vectors, s1f, s2m, idx2,
      Gi, Gj, W1T, b1c, W2T, b2c)

    return scores, fmf != 0.0
